# baseline (speedup 1.0000x reference)
import functools

import jax
import jax.numpy as jnp
from jax import lax
from jax.experimental import pallas as pl
from jax.experimental.pallas import tpu as pltpu
from jax.experimental.pallas import tpu_sc as plsc

N = 10000
E = 320000
D_IN = 128
D_HID = 256
D_OUT = 128

NC = 2
NS = 16
NW = NC * NS
CHUNK = 128
NP = 10112
ROWS_PER_TILE = NP // NS
CPT = 80
IB = 8
E_PAD = NW * CPT * CHUNK


def _make_segsum(D):
    mesh = plsc.VectorSubcoreMesh(core_axis_name="c", subcore_axis_name="s")

    @functools.partial(
        pl.kernel,
        mesh=mesh,
        compiler_params=pltpu.CompilerParams(use_tc_tiling_on_sc=False),
        out_type=jax.ShapeDtypeStruct((NC * NP, D), jnp.float32),
        scratch_types=[
            pltpu.VMEM_SHARED((NP, D), jnp.float32),
            pltpu.VMEM((CHUNK,), jnp.int32),
            pltpu.VMEM((CHUNK,), jnp.int32),
            pltpu.VMEM((CHUNK, D), jnp.float32),
            pltpu.VMEM((CHUNK, D), jnp.float32),
            pltpu.SemaphoreType.DMA,
            pltpu.SemaphoreType.DMA,
        ],
    )
    def segsum(table, srcf, dstf, out, acc, sidx, didx, rows0, rows1,
               semg0, semg1):
        c = lax.axis_index("c")
        s = lax.axis_index("s")
        wid = c * NS + s

        def zero_row(i, carry):
            for j in range(D // 16):
                rows0[i, pl.ds(j * 16, 16)] = jnp.zeros((16,), jnp.float32)
            return carry

        lax.fori_loop(0, CHUNK, zero_row, 0)
        row0 = s * ROWS_PER_TILE
        for k in range(4):
            pltpu.sync_copy(rows0, acc.at[pl.ds(row0 + k * CHUNK, CHUNK)])
        pltpu.sync_copy(rows0.at[pl.ds(0, ROWS_PER_TILE - 4 * CHUNK)],
                        acc.at[pl.ds(row0 + 4 * CHUNK,
                                     ROWS_PER_TILE - 4 * CHUNK)])
        plsc.subcore_barrier()

        def gather(j, buf, sem):
            pltpu.async_copy(table.at[sidx.at[j]], buf, sem)

        def gwait(j, buf, sem):
            pltpu.make_async_copy(table.at[sidx.at[j]], buf, sem).wait()

        def scat(j, buf):
            pltpu.sync_copy(buf, acc.at[didx.at[j]], add=True)

        def block(b, carry):
            e0 = (wid * CPT + b) * CHUNK
            pltpu.sync_copy(srcf.at[pl.ds(e0, CHUNK)], sidx)
            pltpu.sync_copy(dstf.at[pl.ds(e0, CHUNK)], didx)
            pltpu.async_copy(table.at[sidx], rows0, semg0).wait()
            pltpu.sync_copy(rows0, acc.at[didx], add=True)
            return carry

        lax.fori_loop(0, CPT, block, 0)
        plsc.subcore_barrier()

        for k in range(4):
            r = row0 + k * CHUNK
            pltpu.sync_copy(acc.at[pl.ds(r, CHUNK)], rows0)
            pltpu.sync_copy(rows0, out.at[pl.ds(c * NP + r, CHUNK)])
        rem = ROWS_PER_TILE - 4 * CHUNK
        r = row0 + 4 * CHUNK
        pltpu.sync_copy(acc.at[pl.ds(r, rem)], rows0.at[pl.ds(0, rem)])
        pltpu.sync_copy(rows0.at[pl.ds(0, rem)], out.at[pl.ds(c * NP + r, rem)])

    return segsum


_segsum144 = _make_segsum(D_IN + 16)
_segsum128 = _make_segsum(D_OUT)

_R = 1000


def _dense0_body(x_ref, a0_ref, a1_ref, ws0_ref, wn0_ref, b0_ref, wn1_ref,
                 h_ref, p_ref):
    agg = a0_ref[:, :D_IN] + a1_ref[:, :D_IN]
    deg = a0_ref[:, D_IN:D_IN + 16] + a1_ref[:, D_IN:D_IN + 16]
    invd = 1.0 / jnp.clip(deg[:, :1], 1.0, None)
    nb = agg * invd
    h = x_ref[...] @ ws0_ref[...] + nb @ wn0_ref[...] + b0_ref[...]
    h = jnp.maximum(h, 0.0)
    h_ref[...] = h
    p_ref[...] = h @ wn1_ref[...]


def _dense0(x, a0, a1, Ws0, Wn0, b0, Wn1):
    D0 = D_IN + 16
    return pl.pallas_call(
        _dense0_body,
        grid=(N // _R,),
        in_specs=[
            pl.BlockSpec((_R, D_IN), lambda i: (i, 0)),
            pl.BlockSpec((_R, D0), lambda i: (i, 0)),
            pl.BlockSpec((_R, D0), lambda i: (i, 0)),
            pl.BlockSpec((D_IN, D_HID), lambda i: (0, 0)),
            pl.BlockSpec((D_IN, D_HID), lambda i: (0, 0)),
            pl.BlockSpec((1, D_HID), lambda i: (0, 0)),
            pl.BlockSpec((D_HID, D_OUT), lambda i: (0, 0)),
        ],
        out_specs=[
            pl.BlockSpec((_R, D_HID), lambda i: (i, 0)),
            pl.BlockSpec((_R, D_OUT), lambda i: (i, 0)),
        ],
        out_shape=[
            jax.ShapeDtypeStruct((N, D_HID), jnp.float32),
            jax.ShapeDtypeStruct((N, D_OUT), jnp.float32),
        ],
    )(x, a0, a1, Ws0, Wn0, b0, Wn1)


def _dense1_body(h_ref, a0_ref, a1_ref, d0_ref, d1_ref, ws1_ref, b1_ref,
                 o_ref):
    deg = d0_ref[:, :1] + d1_ref[:, :1]
    invd = 1.0 / jnp.clip(deg, 1.0, None)
    nb = (a0_ref[...] + a1_ref[...]) * invd
    o_ref[...] = h_ref[...] @ ws1_ref[...] + nb + b1_ref[...]


def _dense1(h, a0, a1, d0, d1, Ws1, b1):
    return pl.pallas_call(
        _dense1_body,
        grid=(N // _R,),
        in_specs=[
            pl.BlockSpec((_R, D_HID), lambda i: (i, 0)),
            pl.BlockSpec((_R, D_OUT), lambda i: (i, 0)),
            pl.BlockSpec((_R, D_OUT), lambda i: (i, 0)),
            pl.BlockSpec((_R, 16), lambda i: (i, 0)),
            pl.BlockSpec((_R, 16), lambda i: (i, 0)),
            pl.BlockSpec((D_HID, D_OUT), lambda i: (0, 0)),
            pl.BlockSpec((1, D_OUT), lambda i: (0, 0)),
        ],
        out_specs=pl.BlockSpec((_R, D_OUT), lambda i: (i, 0)),
        out_shape=jax.ShapeDtypeStruct((N, D_OUT), jnp.float32),
    )(h, a0, a1, d0, d1, Ws1, b1)


def kernel(x, edge_index, W_self0, W_neigh0, b0, W_self1, W_neigh1, b1):
    src = edge_index[0]
    dst = edge_index[1]
    pad = E_PAD - E
    src2 = jnp.concatenate([src, jnp.zeros((pad,), jnp.int32)])
    dst2 = jnp.concatenate([dst, jnp.full((pad,), NP - 1, jnp.int32)])
    x_ext = jnp.concatenate(
        [x, jnp.ones((N, 16), jnp.float32)], axis=1)
    parts0 = _segsum144(x_ext, src2, dst2)
    a0, a1 = parts0[:N], parts0[NP:NP + N]
    h, p = _dense0(x, a0, a1, W_self0, W_neigh0,
                   b0.reshape(1, -1), W_neigh1)
    parts1 = _segsum128(p, src2, dst2)
    out = _dense1(h, parts1[:N], parts1[NP:NP + N],
                  a0[:, D_IN:D_IN + 16], a1[:, D_IN:D_IN + 16],
                  W_self1, b1.reshape(1, -1))
    return out


# speedup vs baseline: 1.0061x; 1.0061x over previous
import functools

import jax
import jax.numpy as jnp
from jax import lax
from jax.experimental import pallas as pl
from jax.experimental.pallas import tpu as pltpu
from jax.experimental.pallas import tpu_sc as plsc

N = 10000
E = 320000
D_IN = 128
D_HID = 256
D_OUT = 128

NC = 2
NS = 16
NW = NC * NS
CHUNK = 128
NP = 10112
ROWS_PER_TILE = NP // NS
CPT = 80
IB = 8
E_PAD = NW * CPT * CHUNK


def _make_segsum(D):
    mesh = plsc.VectorSubcoreMesh(core_axis_name="c", subcore_axis_name="s")

    @functools.partial(
        pl.kernel,
        mesh=mesh,
        compiler_params=pltpu.CompilerParams(use_tc_tiling_on_sc=False),
        out_type=jax.ShapeDtypeStruct((NC * NP, D), jnp.float32),
        scratch_types=[
            pltpu.VMEM_SHARED((NP, D), jnp.float32),
            pltpu.VMEM((CHUNK,), jnp.int32),
            pltpu.VMEM((CHUNK,), jnp.int32),
            pltpu.VMEM((CHUNK, D), jnp.float32),
            pltpu.VMEM((CHUNK, D), jnp.float32),
            pltpu.SemaphoreType.DMA,
            pltpu.SemaphoreType.DMA,
        ],
    )
    def segsum(table, srcf, dstf, out, acc, sidx, didx, rows0, rows1,
               semg0, semg1):
        c = lax.axis_index("c")
        s = lax.axis_index("s")
        wid = c * NS + s

        def zero_row(i, carry):
            for j in range(D // 16):
                rows0[i, pl.ds(j * 16, 16)] = jnp.zeros((16,), jnp.float32)
            return carry

        lax.fori_loop(0, CHUNK, zero_row, 0)
        row0 = s * ROWS_PER_TILE
        for k in range(4):
            pltpu.sync_copy(rows0, acc.at[pl.ds(row0 + k * CHUNK, CHUNK)])
        pltpu.sync_copy(rows0.at[pl.ds(0, ROWS_PER_TILE - 4 * CHUNK)],
                        acc.at[pl.ds(row0 + 4 * CHUNK,
                                     ROWS_PER_TILE - 4 * CHUNK)])
        plsc.subcore_barrier()

        def gather(j, buf, sem):
            pltpu.async_copy(table.at[sidx.at[j]], buf, sem)

        def gwait(j, buf, sem):
            pltpu.make_async_copy(table.at[sidx.at[j]], buf, sem).wait()

        def scat(j, buf):
            pltpu.sync_copy(buf, acc.at[didx.at[j]], add=True)

        def block(b, carry):
            e0 = (wid * CPT + b) * CHUNK
            pltpu.sync_copy(srcf.at[pl.ds(e0, CHUNK)], sidx)
            pltpu.sync_copy(dstf.at[pl.ds(e0, CHUNK)], didx)
            pltpu.async_copy(table.at[sidx], rows0, semg0).wait()
            pltpu.sync_copy(rows0, acc.at[didx], add=True)
            return carry

        lax.fori_loop(0, CPT, block, 0)
        plsc.subcore_barrier()

        for k in range(4):
            r = row0 + k * CHUNK
            pltpu.sync_copy(acc.at[pl.ds(r, CHUNK)], rows0)
            pltpu.sync_copy(rows0, out.at[pl.ds(c * NP + r, CHUNK)])
        rem = ROWS_PER_TILE - 4 * CHUNK
        r = row0 + 4 * CHUNK
        pltpu.sync_copy(acc.at[pl.ds(r, rem)], rows0.at[pl.ds(0, rem)])
        pltpu.sync_copy(rows0.at[pl.ds(0, rem)], out.at[pl.ds(c * NP + r, rem)])

    return segsum


_segsum144 = _make_segsum(D_IN + 16)
_segsum128 = _make_segsum(D_OUT)

_R = 1000


def _dense0_body(x_ref, a0_ref, a1_ref, ws0_ref, wn0_ref, b0_ref, wn1_ref,
                 h_ref, p_ref):
    agg = a0_ref[:, :D_IN] + a1_ref[:, :D_IN]
    deg = a0_ref[:, D_IN:D_IN + 16] + a1_ref[:, D_IN:D_IN + 16]
    invd = 1.0 / jnp.clip(deg[:, :1], 1.0, None)
    nb = agg * invd
    h = x_ref[...] @ ws0_ref[...] + nb @ wn0_ref[...] + b0_ref[...]
    h = jnp.maximum(h, 0.0)
    h_ref[...] = h
    p_ref[...] = h @ wn1_ref[...]


def _dense0(x, a0, a1, Ws0, Wn0, b0, Wn1):
    D0 = D_IN + 16
    return pl.pallas_call(
        _dense0_body,
        grid=(N // _R,),
        in_specs=[
            pl.BlockSpec((_R, D_IN), lambda i: (i, 0)),
            pl.BlockSpec((_R, D0), lambda i: (i, 0)),
            pl.BlockSpec((_R, D0), lambda i: (i, 0)),
            pl.BlockSpec((D_IN, D_HID), lambda i: (0, 0)),
            pl.BlockSpec((D_IN, D_HID), lambda i: (0, 0)),
            pl.BlockSpec((1, D_HID), lambda i: (0, 0)),
            pl.BlockSpec((D_HID, D_OUT), lambda i: (0, 0)),
        ],
        out_specs=[
            pl.BlockSpec((_R, D_HID), lambda i: (i, 0)),
            pl.BlockSpec((_R, D_OUT), lambda i: (i, 0)),
        ],
        out_shape=[
            jax.ShapeDtypeStruct((N, D_HID), jnp.float32),
            jax.ShapeDtypeStruct((N, D_OUT), jnp.float32),
        ],
    )(x, a0, a1, Ws0, Wn0, b0, Wn1)


def _dense1_body(h_ref, a0_ref, a1_ref, d0_ref, d1_ref, ws1_ref, b1_ref,
                 o_ref):
    deg = d0_ref[:, :1] + d1_ref[:, :1]
    invd = 1.0 / jnp.clip(deg, 1.0, None)
    nb = (a0_ref[...] + a1_ref[...]) * invd
    o_ref[...] = h_ref[...] @ ws1_ref[...] + nb + b1_ref[...]


def _dense1(h, a0, a1, d0, d1, Ws1, b1):
    return pl.pallas_call(
        _dense1_body,
        grid=(N // _R,),
        in_specs=[
            pl.BlockSpec((_R, D_HID), lambda i: (i, 0)),
            pl.BlockSpec((_R, D_OUT), lambda i: (i, 0)),
            pl.BlockSpec((_R, D_OUT), lambda i: (i, 0)),
            pl.BlockSpec((_R, 16), lambda i: (i, 0)),
            pl.BlockSpec((_R, 16), lambda i: (i, 0)),
            pl.BlockSpec((D_HID, D_OUT), lambda i: (0, 0)),
            pl.BlockSpec((1, D_OUT), lambda i: (0, 0)),
        ],
        out_specs=pl.BlockSpec((_R, D_OUT), lambda i: (i, 0)),
        out_shape=jax.ShapeDtypeStruct((N, D_OUT), jnp.float32),
    )(h, a0, a1, d0, d1, Ws1, b1)


def kernel(x, edge_index, W_self0, W_neigh0, b0, W_self1, W_neigh1, b1):
    src = edge_index[0]
    dst = edge_index[1]
    pad = E_PAD - E
    src2 = jnp.concatenate([src, jnp.zeros((pad,), jnp.int32)])
    dst_pad = N + (jnp.arange(pad, dtype=jnp.int32) % (NP - N))
    dst2 = jnp.concatenate([dst, dst_pad])
    x_ext = jnp.concatenate(
        [x, jnp.ones((N, 16), jnp.float32)], axis=1)
    parts0 = _segsum144(x_ext, src2, dst2)
    a0, a1 = parts0[:N], parts0[NP:NP + N]
    h, p = _dense0(x, a0, a1, W_self0, W_neigh0,
                   b0.reshape(1, -1), W_neigh1)
    parts1 = _segsum128(p, src2, dst2)
    out = _dense1(h, parts1[:N], parts1[NP:NP + N],
                  a0[:, D_IN:D_IN + 16], a1[:, D_IN:D_IN + 16],
                  W_self1, b1.reshape(1, -1))
    return out


# speedup vs baseline: 2.2023x; 2.1889x over previous
import functools

import jax
import jax.numpy as jnp
from jax import lax
from jax.experimental import pallas as pl
from jax.experimental.pallas import tpu as pltpu
from jax.experimental.pallas import tpu_sc as plsc

N = 10000
E = 320000
D_IN = 128
D_HID = 256
D_OUT = 128

NC = 2
NS = 16
CHUNK = 128
NP = 10240
ROWS_PER_TILE = NP // NS
ZR = 128
N_COPIES = ROWS_PER_TILE // ZR
CHUNKS_PER_CORE = E // (NC * CHUNK)
ITERS = (CHUNKS_PER_CORE + NS - 1) // NS


def _make_segsum(D):
    mesh = plsc.VectorSubcoreMesh(core_axis_name="c", subcore_axis_name="s")

    @functools.partial(
        pl.kernel,
        mesh=mesh,
        compiler_params=pltpu.CompilerParams(use_tc_tiling_on_sc=False),
        out_type=jax.ShapeDtypeStruct((NC * NP, D), jnp.float32),
        scratch_types=[
            pltpu.VMEM_SHARED((NP, D), jnp.float32),
            pltpu.VMEM((CHUNK,), jnp.int32),
            pltpu.VMEM((CHUNK,), jnp.int32),
            pltpu.VMEM((CHUNK, D), jnp.float32),
            pltpu.VMEM((ZR, D), jnp.float32),
            pltpu.SemaphoreType.DMA,
        ],
    )
    def segsum(table, src, dst, out, acc, sidx, didx, rows, zbuf, sem):
        c = lax.axis_index("c")
        s = lax.axis_index("s")

        def zero_row(i, carry):
            for j in range(D // 16):
                zbuf[i, pl.ds(j * 16, 16)] = jnp.zeros((16,), jnp.float32)
            return carry

        lax.fori_loop(0, ZR, zero_row, 0)
        row0 = s * ROWS_PER_TILE
        for k in range(N_COPIES):
            pltpu.sync_copy(zbuf, acc.at[pl.ds(row0 + k * ZR, ZR)])
        plsc.subcore_barrier()

        def body(j, carry):
            cid = s + j * NS

            @pl.when(cid < CHUNKS_PER_CORE)
            def _():
                e0 = c * (E // NC) + cid * CHUNK
                pltpu.sync_copy(src.at[pl.ds(e0, CHUNK)], sidx)
                pltpu.sync_copy(dst.at[pl.ds(e0, CHUNK)], didx)
                pltpu.async_copy(table.at[sidx], rows, sem).wait()
                pltpu.sync_copy(rows, acc.at[didx], add=True)

            return carry

        lax.fori_loop(0, ITERS, body, 0)
        plsc.subcore_barrier()

        for k in range(N_COPIES):
            r = row0 + k * ZR
            pltpu.sync_copy(acc.at[pl.ds(r, ZR)], zbuf)
            pltpu.sync_copy(zbuf, out.at[pl.ds(c * NP + r, ZR)])

    return segsum


_segsum144 = _make_segsum(D_IN + 16)
_segsum128 = _make_segsum(D_OUT)

_R = 1000


def _dense0_body(x_ref, a0_ref, a1_ref, ws0_ref, wn0_ref, b0_ref, wn1_ref,
                 h_ref, p_ref):
    agg = a0_ref[:, :D_IN] + a1_ref[:, :D_IN]
    deg = a0_ref[:, D_IN:D_IN + 16] + a1_ref[:, D_IN:D_IN + 16]
    invd = 1.0 / jnp.clip(deg[:, :1], 1.0, None)
    nb = agg * invd
    h = x_ref[...] @ ws0_ref[...] + nb @ wn0_ref[...] + b0_ref[...]
    h = jnp.maximum(h, 0.0)
    h_ref[...] = h
    p_ref[...] = h @ wn1_ref[...]


def _dense0(x, a0, a1, Ws0, Wn0, b0, Wn1):
    D0 = D_IN + 16
    return pl.pallas_call(
        _dense0_body,
        grid=(N // _R,),
        in_specs=[
            pl.BlockSpec((_R, D_IN), lambda i: (i, 0)),
            pl.BlockSpec((_R, D0), lambda i: (i, 0)),
            pl.BlockSpec((_R, D0), lambda i: (i, 0)),
            pl.BlockSpec((D_IN, D_HID), lambda i: (0, 0)),
            pl.BlockSpec((D_IN, D_HID), lambda i: (0, 0)),
            pl.BlockSpec((1, D_HID), lambda i: (0, 0)),
            pl.BlockSpec((D_HID, D_OUT), lambda i: (0, 0)),
        ],
        out_specs=[
            pl.BlockSpec((_R, D_HID), lambda i: (i, 0)),
            pl.BlockSpec((_R, D_OUT), lambda i: (i, 0)),
        ],
        out_shape=[
            jax.ShapeDtypeStruct((N, D_HID), jnp.float32),
            jax.ShapeDtypeStruct((N, D_OUT), jnp.float32),
        ],
    )(x, a0, a1, Ws0, Wn0, b0, Wn1)


def _dense1_body(h_ref, a0_ref, a1_ref, d0_ref, d1_ref, ws1_ref, b1_ref,
                 o_ref):
    deg = d0_ref[:, :1] + d1_ref[:, :1]
    invd = 1.0 / jnp.clip(deg, 1.0, None)
    nb = (a0_ref[...] + a1_ref[...]) * invd
    o_ref[...] = h_ref[...] @ ws1_ref[...] + nb + b1_ref[...]


def _dense1(h, a0, a1, d0, d1, Ws1, b1):
    return pl.pallas_call(
        _dense1_body,
        grid=(N // _R,),
        in_specs=[
            pl.BlockSpec((_R, D_HID), lambda i: (i, 0)),
            pl.BlockSpec((_R, D_OUT), lambda i: (i, 0)),
            pl.BlockSpec((_R, D_OUT), lambda i: (i, 0)),
            pl.BlockSpec((_R, 16), lambda i: (i, 0)),
            pl.BlockSpec((_R, 16), lambda i: (i, 0)),
            pl.BlockSpec((D_HID, D_OUT), lambda i: (0, 0)),
            pl.BlockSpec((1, D_OUT), lambda i: (0, 0)),
        ],
        out_specs=pl.BlockSpec((_R, D_OUT), lambda i: (i, 0)),
        out_shape=jax.ShapeDtypeStruct((N, D_OUT), jnp.float32),
    )(h, a0, a1, d0, d1, Ws1, b1)


def kernel(x, edge_index, W_self0, W_neigh0, b0, W_self1, W_neigh1, b1):
    src = edge_index[0]
    dst = edge_index[1]
    x_ext = jnp.concatenate(
        [x, jnp.ones((N, 16), jnp.float32)], axis=1)
    parts0 = _segsum144(x_ext, src, dst)
    a0, a1 = parts0[:N], parts0[NP:NP + N]
    h, p = _dense0(x, a0, a1, W_self0, W_neigh0,
                   b0.reshape(1, -1), W_neigh1)
    parts1 = _segsum128(p, src, dst)
    out = _dense1(h, parts1[:N], parts1[NP:NP + N],
                  a0[:, D_IN:D_IN + 16], a1[:, D_IN:D_IN + 16],
                  W_self1, b1.reshape(1, -1))
    return out


# speedup vs baseline: 3.2535x; 1.4773x over previous
import functools

import jax
import jax.numpy as jnp
from jax import lax
from jax.experimental import pallas as pl
from jax.experimental.pallas import tpu as pltpu
from jax.experimental.pallas import tpu_sc as plsc

N = 10000
E = 320000
D_IN = 128
D_HID = 256
D_OUT = 128

NC = 2
NS = 16
CHUNK = 128
NP = 10240
ROWS_PER_TILE = NP // NS
ZR = 128
N_COPIES = ROWS_PER_TILE // ZR
CHUNKS_PER_CORE = E // (NC * CHUNK)
ITERS = (CHUNKS_PER_CORE + NS - 1) // NS


def _make_segsum(D):
    mesh = plsc.VectorSubcoreMesh(core_axis_name="c", subcore_axis_name="s")

    @functools.partial(
        pl.kernel,
        mesh=mesh,
        compiler_params=pltpu.CompilerParams(use_tc_tiling_on_sc=False),
        out_type=jax.ShapeDtypeStruct((NC * NP, D), jnp.float32),
        scratch_types=[
            pltpu.VMEM_SHARED((NP, D), jnp.float32),
            pltpu.VMEM((CHUNK,), jnp.int32),
            pltpu.VMEM((CHUNK,), jnp.int32),
            pltpu.VMEM((CHUNK,), jnp.int32),
            pltpu.VMEM((CHUNK,), jnp.int32),
            pltpu.VMEM((CHUNK, D), jnp.float32),
            pltpu.VMEM((ZR, D), jnp.float32),
            pltpu.SemaphoreType.DMA,
            pltpu.SemaphoreType.DMA,
        ],
    )
    def segsum(table, src, dst, out, acc, sidx0, didx0, sidx1, didx1,
               rows, zbuf, sem0, sem1):
        c = lax.axis_index("c")
        s = lax.axis_index("s")

        def zero_row(i, carry):
            for j in range(D // 16):
                zbuf[i, pl.ds(j * 16, 16)] = jnp.zeros((16,), jnp.float32)
            return carry

        lax.fori_loop(0, ZR, zero_row, 0)
        row0 = s * ROWS_PER_TILE
        for k in range(N_COPIES):
            pltpu.sync_copy(zbuf, acc.at[pl.ds(row0 + k * ZR, ZR)])
        plsc.subcore_barrier()

        def load_idx(cid, sidx_b, didx_b):
            e0 = c * (E // NC) + cid * CHUNK
            pltpu.sync_copy(src.at[pl.ds(e0, CHUNK)], sidx_b)
            pltpu.sync_copy(dst.at[pl.ds(e0, CHUNK)], didx_b)

        load_idx(s, sidx0, didx0)
        pltpu.async_copy(table.at[sidx0], rows, sem0)

        def body(jj, carry):
            cid0 = s + (2 * jj) * NS
            cid1 = s + (2 * jj + 1) * NS
            cid2 = s + (2 * jj + 2) * NS

            @pl.when(cid1 < CHUNKS_PER_CORE)
            def _():
                load_idx(cid1, sidx1, didx1)
                pltpu.async_copy(table.at[sidx1], zbuf, sem1)

            @pl.when(cid0 < CHUNKS_PER_CORE)
            def _():
                pltpu.make_async_copy(table.at[sidx0], rows, sem0).wait()
                pltpu.sync_copy(rows, acc.at[didx0], add=True)

            @pl.when(cid2 < CHUNKS_PER_CORE)
            def _():
                load_idx(cid2, sidx0, didx0)
                pltpu.async_copy(table.at[sidx0], rows, sem0)

            @pl.when(cid1 < CHUNKS_PER_CORE)
            def _():
                pltpu.make_async_copy(table.at[sidx1], zbuf, sem1).wait()
                pltpu.sync_copy(zbuf, acc.at[didx1], add=True)

            return carry

        lax.fori_loop(0, (ITERS + 1) // 2, body, 0)
        plsc.subcore_barrier()

        for k in range(N_COPIES):
            r = row0 + k * ZR
            pltpu.sync_copy(acc.at[pl.ds(r, ZR)], zbuf)
            pltpu.sync_copy(zbuf, out.at[pl.ds(c * NP + r, ZR)])

    return segsum


_segsum144 = _make_segsum(D_IN + 16)
_segsum128 = _make_segsum(D_OUT)

_R = 1000


def _dense0_body(x_ref, a0_ref, a1_ref, ws0_ref, wn0_ref, b0_ref, wn1_ref,
                 h_ref, p_ref):
    agg = a0_ref[:, :D_IN] + a1_ref[:, :D_IN]
    deg = a0_ref[:, D_IN:D_IN + 16] + a1_ref[:, D_IN:D_IN + 16]
    invd = 1.0 / jnp.clip(deg[:, :1], 1.0, None)
    nb = agg * invd
    h = x_ref[...] @ ws0_ref[...] + nb @ wn0_ref[...] + b0_ref[...]
    h = jnp.maximum(h, 0.0)
    h_ref[...] = h
    p_ref[...] = h @ wn1_ref[...]


def _dense0(x, a0, a1, Ws0, Wn0, b0, Wn1):
    D0 = D_IN + 16
    return pl.pallas_call(
        _dense0_body,
        grid=(N // _R,),
        in_specs=[
            pl.BlockSpec((_R, D_IN), lambda i: (i, 0)),
            pl.BlockSpec((_R, D0), lambda i: (i, 0)),
            pl.BlockSpec((_R, D0), lambda i: (i, 0)),
            pl.BlockSpec((D_IN, D_HID), lambda i: (0, 0)),
            pl.BlockSpec((D_IN, D_HID), lambda i: (0, 0)),
            pl.BlockSpec((1, D_HID), lambda i: (0, 0)),
            pl.BlockSpec((D_HID, D_OUT), lambda i: (0, 0)),
        ],
        out_specs=[
            pl.BlockSpec((_R, D_HID), lambda i: (i, 0)),
            pl.BlockSpec((_R, D_OUT), lambda i: (i, 0)),
        ],
        out_shape=[
            jax.ShapeDtypeStruct((N, D_HID), jnp.float32),
            jax.ShapeDtypeStruct((N, D_OUT), jnp.float32),
        ],
    )(x, a0, a1, Ws0, Wn0, b0, Wn1)


def _dense1_body(h_ref, a0_ref, a1_ref, d0_ref, d1_ref, ws1_ref, b1_ref,
                 o_ref):
    deg = d0_ref[:, :1] + d1_ref[:, :1]
    invd = 1.0 / jnp.clip(deg, 1.0, None)
    nb = (a0_ref[...] + a1_ref[...]) * invd
    o_ref[...] = h_ref[...] @ ws1_ref[...] + nb + b1_ref[...]


def _dense1(h, a0, a1, d0, d1, Ws1, b1):
    return pl.pallas_call(
        _dense1_body,
        grid=(N // _R,),
        in_specs=[
            pl.BlockSpec((_R, D_HID), lambda i: (i, 0)),
            pl.BlockSpec((_R, D_OUT), lambda i: (i, 0)),
            pl.BlockSpec((_R, D_OUT), lambda i: (i, 0)),
            pl.BlockSpec((_R, 16), lambda i: (i, 0)),
            pl.BlockSpec((_R, 16), lambda i: (i, 0)),
            pl.BlockSpec((D_HID, D_OUT), lambda i: (0, 0)),
            pl.BlockSpec((1, D_OUT), lambda i: (0, 0)),
        ],
        out_specs=pl.BlockSpec((_R, D_OUT), lambda i: (i, 0)),
        out_shape=jax.ShapeDtypeStruct((N, D_OUT), jnp.float32),
    )(h, a0, a1, d0, d1, Ws1, b1)


def kernel(x, edge_index, W_self0, W_neigh0, b0, W_self1, W_neigh1, b1):
    src = edge_index[0]
    dst = edge_index[1]
    x_ext = jnp.concatenate(
        [x, jnp.ones((N, 16), jnp.float32)], axis=1)
    parts0 = _segsum144(x_ext, src, dst)
    a0, a1 = parts0[:N], parts0[NP:NP + N]
    h, p = _dense0(x, a0, a1, W_self0, W_neigh0,
                   b0.reshape(1, -1), W_neigh1)
    parts1 = _segsum128(p, src, dst)
    out = _dense1(h, parts1[:N], parts1[NP:NP + N],
                  a0[:, D_IN:D_IN + 16], a1[:, D_IN:D_IN + 16],
                  W_self1, b1.reshape(1, -1))
    return out


# speedup vs baseline: 3.2650x; 1.0035x over previous
import functools

import jax
import jax.numpy as jnp
from jax import lax
from jax.experimental import pallas as pl
from jax.experimental.pallas import tpu as pltpu
from jax.experimental.pallas import tpu_sc as plsc

N = 10000
E = 320000
D_IN = 128
D_HID = 256
D_OUT = 128

NC = 2
NS = 16
CHUNK = 128
NP = 10240
ROWS_PER_TILE = NP // NS
ZR = 128
N_COPIES = ROWS_PER_TILE // ZR
CHUNKS_PER_CORE = E // (NC * CHUNK)
ITERS = (CHUNKS_PER_CORE + NS - 1) // NS


def _make_segsum(D):
    mesh = plsc.VectorSubcoreMesh(core_axis_name="c", subcore_axis_name="s")

    @functools.partial(
        pl.kernel,
        mesh=mesh,
        compiler_params=pltpu.CompilerParams(use_tc_tiling_on_sc=False),
        out_type=jax.ShapeDtypeStruct((NC * NP, D), jnp.float32),
        scratch_types=[
            pltpu.VMEM_SHARED((NP, D), jnp.float32),
            pltpu.VMEM((CHUNK,), jnp.int32),
            pltpu.VMEM((CHUNK,), jnp.int32),
            pltpu.VMEM((CHUNK,), jnp.int32),
            pltpu.VMEM((CHUNK,), jnp.int32),
            pltpu.VMEM((CHUNK, D), jnp.float32),
            pltpu.VMEM((ZR, D), jnp.float32),
            pltpu.SemaphoreType.DMA,
            pltpu.SemaphoreType.DMA,
        ],
    )
    def segsum(table, src, dst, out, acc, sidx0, didx0, sidx1, didx1,
               rows, zbuf, sem0, sem1):
        c = lax.axis_index("c")
        s = lax.axis_index("s")

        def zero_row(i, carry):
            for j in range(D // 16):
                zbuf[i, pl.ds(j * 16, 16)] = jnp.zeros((16,), jnp.float32)
            return carry

        lax.fori_loop(0, ZR, zero_row, 0)
        row0 = s * ROWS_PER_TILE
        for k in range(N_COPIES):
            pltpu.sync_copy(zbuf, acc.at[pl.ds(row0 + k * ZR, ZR)])
        plsc.subcore_barrier()

        def load_idx(cid, sidx_b, didx_b):
            e0 = c * (E // NC) + cid * CHUNK
            pltpu.sync_copy(src.at[pl.ds(e0, CHUNK)], sidx_b)
            pltpu.sync_copy(dst.at[pl.ds(e0, CHUNK)], didx_b)

        load_idx(s, sidx0, didx0)
        pltpu.async_copy(table.at[sidx0], rows, sem0)

        def body(jj, carry):
            cid0 = s + (2 * jj) * NS
            cid1 = s + (2 * jj + 1) * NS
            cid2 = s + (2 * jj + 2) * NS

            @pl.when(cid1 < CHUNKS_PER_CORE)
            def _():
                load_idx(cid1, sidx1, didx1)
                pltpu.async_copy(table.at[sidx1], zbuf, sem1)

            @pl.when(cid0 < CHUNKS_PER_CORE)
            def _():
                pltpu.make_async_copy(table.at[sidx0], rows, sem0).wait()
                pltpu.sync_copy(rows, acc.at[didx0], add=True)

            @pl.when(cid2 < CHUNKS_PER_CORE)
            def _():
                load_idx(cid2, sidx0, didx0)
                pltpu.async_copy(table.at[sidx0], rows, sem0)

            @pl.when(cid1 < CHUNKS_PER_CORE)
            def _():
                pltpu.make_async_copy(table.at[sidx1], zbuf, sem1).wait()
                pltpu.sync_copy(zbuf, acc.at[didx1], add=True)

            return carry

        lax.fori_loop(0, (ITERS + 1) // 2, body, 0)
        plsc.subcore_barrier()

        for k in range(N_COPIES):
            r = row0 + k * ZR
            pltpu.sync_copy(acc.at[pl.ds(r, ZR)], zbuf)
            pltpu.sync_copy(zbuf, out.at[pl.ds(c * NP + r, ZR)])

    return segsum


_segsum144 = _make_segsum(D_IN + 16)
_segsum128 = _make_segsum(D_OUT)

_R = 1000


def _pre_body(x_ref, ws0_ref, b0_ref, hs_ref):
    hs_ref[...] = x_ref[...] @ ws0_ref[...] + b0_ref[...]


def _pre(x, Ws0, b0):
    return pl.pallas_call(
        _pre_body,
        grid=(N // _R,),
        in_specs=[
            pl.BlockSpec((_R, D_IN), lambda i: (i, 0)),
            pl.BlockSpec((D_IN, D_HID), lambda i: (0, 0)),
            pl.BlockSpec((1, D_HID), lambda i: (0, 0)),
        ],
        out_specs=pl.BlockSpec((_R, D_HID), lambda i: (i, 0)),
        out_shape=jax.ShapeDtypeStruct((N, D_HID), jnp.float32),
    )(x, Ws0, b0)


def _dense0_body(hs_ref, a0_ref, a1_ref, wn0_ref, wn1_ref, ws1_ref, b1_ref,
                 p_ref, q_ref):
    agg = a0_ref[:, :D_IN] + a1_ref[:, :D_IN]
    deg = a0_ref[:, D_IN:D_IN + 16] + a1_ref[:, D_IN:D_IN + 16]
    invd = 1.0 / jnp.clip(deg[:, :1], 1.0, None)
    nb = agg * invd
    h = jnp.maximum(hs_ref[...] + nb @ wn0_ref[...], 0.0)
    p_ref[...] = h @ wn1_ref[...]
    q_ref[...] = h @ ws1_ref[...] + b1_ref[...]


def _dense0(hs, a0, a1, Wn0, Wn1, Ws1, b1):
    D0 = D_IN + 16
    return pl.pallas_call(
        _dense0_body,
        grid=(N // _R,),
        in_specs=[
            pl.BlockSpec((_R, D_HID), lambda i: (i, 0)),
            pl.BlockSpec((_R, D0), lambda i: (i, 0)),
            pl.BlockSpec((_R, D0), lambda i: (i, 0)),
            pl.BlockSpec((D_IN, D_HID), lambda i: (0, 0)),
            pl.BlockSpec((D_HID, D_OUT), lambda i: (0, 0)),
            pl.BlockSpec((D_HID, D_OUT), lambda i: (0, 0)),
            pl.BlockSpec((1, D_OUT), lambda i: (0, 0)),
        ],
        out_specs=[
            pl.BlockSpec((_R, D_OUT), lambda i: (i, 0)),
            pl.BlockSpec((_R, D_OUT), lambda i: (i, 0)),
        ],
        out_shape=[
            jax.ShapeDtypeStruct((N, D_OUT), jnp.float32),
            jax.ShapeDtypeStruct((N, D_OUT), jnp.float32),
        ],
    )(hs, a0, a1, Wn0, Wn1, Ws1, b1)


def _dense1_body(q_ref, a0_ref, a1_ref, d0_ref, d1_ref, o_ref):
    deg = d0_ref[:, :1] + d1_ref[:, :1]
    invd = 1.0 / jnp.clip(deg, 1.0, None)
    o_ref[...] = q_ref[...] + (a0_ref[...] + a1_ref[...]) * invd


def _dense1(q, a0, a1, d0, d1):
    return pl.pallas_call(
        _dense1_body,
        grid=(N // _R,),
        in_specs=[
            pl.BlockSpec((_R, D_OUT), lambda i: (i, 0)),
            pl.BlockSpec((_R, D_OUT), lambda i: (i, 0)),
            pl.BlockSpec((_R, D_OUT), lambda i: (i, 0)),
            pl.BlockSpec((_R, 16), lambda i: (i, 0)),
            pl.BlockSpec((_R, 16), lambda i: (i, 0)),
        ],
        out_specs=pl.BlockSpec((_R, D_OUT), lambda i: (i, 0)),
        out_shape=jax.ShapeDtypeStruct((N, D_OUT), jnp.float32),
    )(q, a0, a1, d0, d1)


def kernel(x, edge_index, W_self0, W_neigh0, b0, W_self1, W_neigh1, b1):
    src = edge_index[0]
    dst = edge_index[1]
    x_ext = jnp.concatenate(
        [x, jnp.ones((N, 16), jnp.float32)], axis=1)
    parts0 = _segsum144(x_ext, src, dst)
    hs = _pre(x, W_self0, b0.reshape(1, -1))
    a0, a1 = parts0[:N], parts0[NP:NP + N]
    p, q = _dense0(hs, a0, a1, W_neigh0, W_neigh1, W_self1,
                   b1.reshape(1, -1))
    parts1 = _segsum128(p, src, dst)
    out = _dense1(q, parts1[:N], parts1[NP:NP + N],
                  a0[:, D_IN:D_IN + 16], a1[:, D_IN:D_IN + 16])
    return out


# speedup vs baseline: 3.5909x; 1.0998x over previous
import functools

import jax
import jax.numpy as jnp
from jax import lax
from jax.experimental import pallas as pl
from jax.experimental.pallas import tpu as pltpu
from jax.experimental.pallas import tpu_sc as plsc

N = 10000
E = 320000
D_IN = 128
D_HID = 256
D_OUT = 128

NC = 2
NS = 16
CHUNK = 128
NP = 10240
ROWS_PER_TILE = NP // NS
ZR = 128
N_COPIES = ROWS_PER_TILE // ZR
CHUNKS_PER_CORE = E // (NC * CHUNK)
ITERS = (CHUNKS_PER_CORE + NS - 1) // NS


def _make_segsum(D):
    mesh = plsc.VectorSubcoreMesh(core_axis_name="c", subcore_axis_name="s")

    @functools.partial(
        pl.kernel,
        mesh=mesh,
        compiler_params=pltpu.CompilerParams(use_tc_tiling_on_sc=False),
        out_type=jax.ShapeDtypeStruct((NC * NP, D), jnp.float32),
        scratch_types=[
            pltpu.VMEM_SHARED((NP, D), jnp.float32),
            pltpu.VMEM((CHUNK,), jnp.int32),
            pltpu.VMEM((CHUNK,), jnp.int32),
            pltpu.VMEM((CHUNK,), jnp.int32),
            pltpu.VMEM((CHUNK,), jnp.int32),
            pltpu.VMEM((CHUNK, D), jnp.float32),
            pltpu.VMEM((ZR, D), jnp.float32),
            pltpu.SemaphoreType.DMA,
            pltpu.SemaphoreType.DMA,
            pltpu.SemaphoreType.DMA,
            pltpu.SemaphoreType.DMA,
        ],
    )
    def segsum(table, src, dst, out, acc, sidx0, didx0, sidx1, didx1,
               rows, zbuf, sem0, sem1, semi0, semi1):
        c = lax.axis_index("c")
        s = lax.axis_index("s")

        def zero_row(i, carry):
            for j in range(D // 16):
                zbuf[i, pl.ds(j * 16, 16)] = jnp.zeros((16,), jnp.float32)
            return carry

        lax.fori_loop(0, ZR, zero_row, 0)
        row0 = s * ROWS_PER_TILE
        for k in range(N_COPIES):
            pltpu.sync_copy(zbuf, acc.at[pl.ds(row0 + k * ZR, ZR)])
        plsc.subcore_barrier()

        def idx_start(cid, sidx_b, didx_b, semi_b):
            e0 = c * (E // NC) + cid * CHUNK
            pltpu.async_copy(src.at[pl.ds(e0, CHUNK)], sidx_b, semi_b)
            pltpu.async_copy(dst.at[pl.ds(e0, CHUNK)], didx_b, semi_b)

        def idx_wait(cid, sidx_b, didx_b, semi_b):
            e0 = c * (E // NC) + cid * CHUNK
            pltpu.make_async_copy(src.at[pl.ds(e0, CHUNK)], sidx_b,
                                  semi_b).wait()
            pltpu.make_async_copy(dst.at[pl.ds(e0, CHUNK)], didx_b,
                                  semi_b).wait()

        idx_start(s, sidx0, didx0, semi0)
        idx_wait(s, sidx0, didx0, semi0)
        pltpu.async_copy(table.at[sidx0], rows, sem0)
        idx_start(s + NS, sidx1, didx1, semi1)

        def body(jj, carry):
            cid0 = s + (2 * jj) * NS
            cid1 = s + (2 * jj + 1) * NS
            cid2 = s + (2 * jj + 2) * NS
            cid3 = s + (2 * jj + 3) * NS

            @pl.when(cid0 < CHUNKS_PER_CORE)
            def _():
                pltpu.make_async_copy(table.at[sidx0], rows, sem0).wait()

            @pl.when(cid1 < CHUNKS_PER_CORE)
            def _():
                idx_wait(cid1, sidx1, didx1, semi1)
                pltpu.async_copy(table.at[sidx1], zbuf, sem1)

            @pl.when(cid0 < CHUNKS_PER_CORE)
            def _():
                pltpu.sync_copy(rows, acc.at[didx0], add=True)

            @pl.when(cid2 < CHUNKS_PER_CORE)
            def _():
                idx_start(cid2, sidx0, didx0, semi0)

            @pl.when(cid1 < CHUNKS_PER_CORE)
            def _():
                pltpu.make_async_copy(table.at[sidx1], zbuf, sem1).wait()

            @pl.when(cid2 < CHUNKS_PER_CORE)
            def _():
                idx_wait(cid2, sidx0, didx0, semi0)
                pltpu.async_copy(table.at[sidx0], rows, sem0)

            @pl.when(cid1 < CHUNKS_PER_CORE)
            def _():
                pltpu.sync_copy(zbuf, acc.at[didx1], add=True)

            @pl.when(cid3 < CHUNKS_PER_CORE)
            def _():
                idx_start(cid3, sidx1, didx1, semi1)

            return carry

        lax.fori_loop(0, (ITERS + 1) // 2, body, 0)
        plsc.subcore_barrier()

        for k in range(N_COPIES):
            r = row0 + k * ZR
            pltpu.sync_copy(acc.at[pl.ds(r, ZR)], zbuf)
            pltpu.sync_copy(zbuf, out.at[pl.ds(c * NP + r, ZR)])

    return segsum


_segsum144 = _make_segsum(D_IN + 16)
_segsum128 = _make_segsum(D_OUT)

_R = 1000


def _pre_body(x_ref, ws0_ref, b0_ref, hs_ref):
    hs_ref[...] = x_ref[...] @ ws0_ref[...] + b0_ref[...]


def _pre(x, Ws0, b0):
    return pl.pallas_call(
        _pre_body,
        grid=(N // _R,),
        in_specs=[
            pl.BlockSpec((_R, D_IN), lambda i: (i, 0)),
            pl.BlockSpec((D_IN, D_HID), lambda i: (0, 0)),
            pl.BlockSpec((1, D_HID), lambda i: (0, 0)),
        ],
        out_specs=pl.BlockSpec((_R, D_HID), lambda i: (i, 0)),
        out_shape=jax.ShapeDtypeStruct((N, D_HID), jnp.float32),
    )(x, Ws0, b0)


def _dense0_body(hs_ref, a0_ref, a1_ref, wn0_ref, wn1_ref, ws1_ref, b1_ref,
                 p_ref, q_ref):
    agg = a0_ref[:, :D_IN] + a1_ref[:, :D_IN]
    deg = a0_ref[:, D_IN:D_IN + 16] + a1_ref[:, D_IN:D_IN + 16]
    invd = 1.0 / jnp.clip(deg[:, :1], 1.0, None)
    nb = agg * invd
    h = jnp.maximum(hs_ref[...] + nb @ wn0_ref[...], 0.0)
    p_ref[...] = h @ wn1_ref[...]
    q_ref[...] = h @ ws1_ref[...] + b1_ref[...]


def _dense0(hs, a0, a1, Wn0, Wn1, Ws1, b1):
    D0 = D_IN + 16
    return pl.pallas_call(
        _dense0_body,
        grid=(N // _R,),
        in_specs=[
            pl.BlockSpec((_R, D_HID), lambda i: (i, 0)),
            pl.BlockSpec((_R, D0), lambda i: (i, 0)),
            pl.BlockSpec((_R, D0), lambda i: (i, 0)),
            pl.BlockSpec((D_IN, D_HID), lambda i: (0, 0)),
            pl.BlockSpec((D_HID, D_OUT), lambda i: (0, 0)),
            pl.BlockSpec((D_HID, D_OUT), lambda i: (0, 0)),
            pl.BlockSpec((1, D_OUT), lambda i: (0, 0)),
        ],
        out_specs=[
            pl.BlockSpec((_R, D_OUT), lambda i: (i, 0)),
            pl.BlockSpec((_R, D_OUT), lambda i: (i, 0)),
        ],
        out_shape=[
            jax.ShapeDtypeStruct((N, D_OUT), jnp.float32),
            jax.ShapeDtypeStruct((N, D_OUT), jnp.float32),
        ],
    )(hs, a0, a1, Wn0, Wn1, Ws1, b1)


def _dense1_body(q_ref, a0_ref, a1_ref, d0_ref, d1_ref, o_ref):
    deg = d0_ref[:, :1] + d1_ref[:, :1]
    invd = 1.0 / jnp.clip(deg, 1.0, None)
    o_ref[...] = q_ref[...] + (a0_ref[...] + a1_ref[...]) * invd


def _dense1(q, a0, a1, d0, d1):
    return pl.pallas_call(
        _dense1_body,
        grid=(N // _R,),
        in_specs=[
            pl.BlockSpec((_R, D_OUT), lambda i: (i, 0)),
            pl.BlockSpec((_R, D_OUT), lambda i: (i, 0)),
            pl.BlockSpec((_R, D_OUT), lambda i: (i, 0)),
            pl.BlockSpec((_R, 16), lambda i: (i, 0)),
            pl.BlockSpec((_R, 16), lambda i: (i, 0)),
        ],
        out_specs=pl.BlockSpec((_R, D_OUT), lambda i: (i, 0)),
        out_shape=jax.ShapeDtypeStruct((N, D_OUT), jnp.float32),
    )(q, a0, a1, d0, d1)


def kernel(x, edge_index, W_self0, W_neigh0, b0, W_self1, W_neigh1, b1):
    src = edge_index[0]
    dst = edge_index[1]
    x_ext = jnp.concatenate(
        [x, jnp.ones((N, 16), jnp.float32)], axis=1)
    parts0 = _segsum144(x_ext, src, dst)
    hs = _pre(x, W_self0, b0.reshape(1, -1))
    a0, a1 = parts0[:N], parts0[NP:NP + N]
    p, q = _dense0(hs, a0, a1, W_neigh0, W_neigh1, W_self1,
                   b1.reshape(1, -1))
    parts1 = _segsum128(p, src, dst)
    out = _dense1(q, parts1[:N], parts1[NP:NP + N],
                  a0[:, D_IN:D_IN + 16], a1[:, D_IN:D_IN + 16])
    return out


# speedup vs baseline: 3.7901x; 1.0555x over previous
import functools

import jax
import jax.numpy as jnp
from jax import lax
from jax.experimental import pallas as pl
from jax.experimental.pallas import tpu as pltpu
from jax.experimental.pallas import tpu_sc as plsc

N = 10000
E = 320000
D_IN = 128
D_HID = 256
D_OUT = 128

NC = 2
NS = 16
CHUNK = 128
NP = 10240
ROWS_PER_TILE = NP // NS
ZR = 128
N_COPIES = ROWS_PER_TILE // ZR
CHUNKS_PER_CORE = E // (NC * CHUNK)
ITERS = (CHUNKS_PER_CORE + NS - 1) // NS


def _make_segsum(D):
    mesh = plsc.VectorSubcoreMesh(core_axis_name="c", subcore_axis_name="s")

    @functools.partial(
        pl.kernel,
        mesh=mesh,
        compiler_params=pltpu.CompilerParams(use_tc_tiling_on_sc=False),
        out_type=jax.ShapeDtypeStruct((NC * NP, D), jnp.float32),
        scratch_types=[
            pltpu.VMEM_SHARED((NP, D), jnp.float32),
            pltpu.VMEM((CHUNK,), jnp.int32),
            pltpu.VMEM((CHUNK,), jnp.int32),
            pltpu.VMEM((CHUNK,), jnp.int32),
            pltpu.VMEM((CHUNK,), jnp.int32),
            pltpu.VMEM((CHUNK, D), jnp.float32),
            pltpu.VMEM((ZR, D), jnp.float32),
            pltpu.SemaphoreType.DMA,
            pltpu.SemaphoreType.DMA,
            pltpu.SemaphoreType.DMA,
            pltpu.SemaphoreType.DMA,
        ],
    )
    def segsum(table, src, dst, out, acc, sidx0, didx0, sidx1, didx1,
               rows, zbuf, sem0, sem1, semi0, semi1):
        c = lax.axis_index("c")
        s = lax.axis_index("s")

        def zero_row(i, carry):
            for j in range(D // 16):
                zbuf[i, pl.ds(j * 16, 16)] = jnp.zeros((16,), jnp.float32)
            return carry

        lax.fori_loop(0, ZR, zero_row, 0)
        row0 = s * ROWS_PER_TILE
        for k in range(N_COPIES):
            pltpu.sync_copy(zbuf, acc.at[pl.ds(row0 + k * ZR, ZR)])
        plsc.subcore_barrier()

        def idx_start(cid, sidx_b, didx_b, semi_b):
            e0 = c * (E // NC) + cid * CHUNK
            pltpu.async_copy(src.at[pl.ds(e0, CHUNK)], sidx_b, semi_b)
            pltpu.async_copy(dst.at[pl.ds(e0, CHUNK)], didx_b, semi_b)

        def idx_wait(cid, sidx_b, didx_b, semi_b):
            e0 = c * (E // NC) + cid * CHUNK
            pltpu.make_async_copy(src.at[pl.ds(e0, CHUNK)], sidx_b,
                                  semi_b).wait()
            pltpu.make_async_copy(dst.at[pl.ds(e0, CHUNK)], didx_b,
                                  semi_b).wait()

        idx_start(s, sidx0, didx0, semi0)
        idx_wait(s, sidx0, didx0, semi0)
        pltpu.async_copy(table.at[sidx0], rows, sem0)
        idx_start(s + NS, sidx1, didx1, semi1)

        def body(jj, carry):
            cid0 = s + (2 * jj) * NS
            cid1 = s + (2 * jj + 1) * NS
            cid2 = s + (2 * jj + 2) * NS
            cid3 = s + (2 * jj + 3) * NS

            @pl.when(cid0 < CHUNKS_PER_CORE)
            def _():
                pltpu.make_async_copy(table.at[sidx0], rows, sem0).wait()

            @pl.when(cid1 < CHUNKS_PER_CORE)
            def _():
                idx_wait(cid1, sidx1, didx1, semi1)
                pltpu.async_copy(table.at[sidx1], zbuf, sem1)

            @pl.when(cid0 < CHUNKS_PER_CORE)
            def _():
                pltpu.sync_copy(rows, acc.at[didx0], add=True)

            @pl.when(cid2 < CHUNKS_PER_CORE)
            def _():
                idx_start(cid2, sidx0, didx0, semi0)

            @pl.when(cid1 < CHUNKS_PER_CORE)
            def _():
                pltpu.make_async_copy(table.at[sidx1], zbuf, sem1).wait()

            @pl.when(cid2 < CHUNKS_PER_CORE)
            def _():
                idx_wait(cid2, sidx0, didx0, semi0)
                pltpu.async_copy(table.at[sidx0], rows, sem0)

            @pl.when(cid1 < CHUNKS_PER_CORE)
            def _():
                pltpu.sync_copy(zbuf, acc.at[didx1], add=True)

            @pl.when(cid3 < CHUNKS_PER_CORE)
            def _():
                idx_start(cid3, sidx1, didx1, semi1)

            return carry

        lax.fori_loop(0, (ITERS + 1) // 2, body, 0)
        plsc.subcore_barrier()

        for k in range(N_COPIES):
            r = row0 + k * ZR
            pltpu.sync_copy(acc.at[pl.ds(r, ZR)], zbuf)
            pltpu.sync_copy(zbuf, out.at[pl.ds(c * NP + r, ZR)])

    return segsum


_segsum144 = _make_segsum(D_IN + 16)
_segsum128 = _make_segsum(D_OUT)

_R = 1000


def _pre_body(x_ref, ws0_ref, b0_ref, hs_ref):
    hs_ref[...] = x_ref[...] @ ws0_ref[...] + b0_ref[...]


def _pre(x, Ws0, b0):
    return pl.pallas_call(
        _pre_body,
        grid=(N // _R,),
        in_specs=[
            pl.BlockSpec((_R, D_IN), lambda i: (i, 0)),
            pl.BlockSpec((D_IN, D_HID), lambda i: (0, 0)),
            pl.BlockSpec((1, D_HID), lambda i: (0, 0)),
        ],
        out_specs=pl.BlockSpec((_R, D_HID), lambda i: (i, 0)),
        out_shape=jax.ShapeDtypeStruct((N, D_HID), jnp.float32),
    )(x, Ws0, b0)


def _dense0_body(hs_ref, a0_ref, a1_ref, wn0_ref, wn1_ref, ws1_ref, b1_ref,
                 p_ref, q_ref):
    a0 = a0_ref[0]
    a1 = a1_ref[0]
    agg = a0[:, :D_IN] + a1[:, :D_IN]
    deg = a0[:, D_IN:D_IN + 16] + a1[:, D_IN:D_IN + 16]
    invd = 1.0 / jnp.clip(deg[:, :1], 1.0, None)
    nb = agg * invd
    h = jnp.maximum(hs_ref[...] + nb @ wn0_ref[...], 0.0)
    p_ref[...] = h @ wn1_ref[...]
    q_ref[...] = h @ ws1_ref[...] + b1_ref[...]


def _dense0(hs, parts0, Wn0, Wn1, Ws1, b1):
    D0 = D_IN + 16
    return pl.pallas_call(
        _dense0_body,
        grid=(N // _R,),
        in_specs=[
            pl.BlockSpec((_R, D_HID), lambda i: (i, 0)),
            pl.BlockSpec((1, _R, D0), lambda i: (0, i, 0)),
            pl.BlockSpec((1, _R, D0), lambda i: (1, i, 0)),
            pl.BlockSpec((D_IN, D_HID), lambda i: (0, 0)),
            pl.BlockSpec((D_HID, D_OUT), lambda i: (0, 0)),
            pl.BlockSpec((D_HID, D_OUT), lambda i: (0, 0)),
            pl.BlockSpec((1, D_OUT), lambda i: (0, 0)),
        ],
        out_specs=[
            pl.BlockSpec((_R, D_OUT), lambda i: (i, 0)),
            pl.BlockSpec((_R, D_OUT), lambda i: (i, 0)),
        ],
        out_shape=[
            jax.ShapeDtypeStruct((N, D_OUT), jnp.float32),
            jax.ShapeDtypeStruct((N, D_OUT), jnp.float32),
        ],
    )(hs, parts0, parts0, Wn0, Wn1, Ws1, b1)


def _dense1_body(q_ref, a0_ref, a1_ref, d0_ref, d1_ref, o_ref):
    deg = d0_ref[0, :, D_IN:D_IN + 1] + d1_ref[0, :, D_IN:D_IN + 1]
    invd = 1.0 / jnp.clip(deg, 1.0, None)
    o_ref[...] = q_ref[...] + (a0_ref[0] + a1_ref[0]) * invd


def _dense1(q, parts1, parts0):
    D0 = D_IN + 16
    return pl.pallas_call(
        _dense1_body,
        grid=(N // _R,),
        in_specs=[
            pl.BlockSpec((_R, D_OUT), lambda i: (i, 0)),
            pl.BlockSpec((1, _R, D_OUT), lambda i: (0, i, 0)),
            pl.BlockSpec((1, _R, D_OUT), lambda i: (1, i, 0)),
            pl.BlockSpec((1, _R, D0), lambda i: (0, i, 0)),
            pl.BlockSpec((1, _R, D0), lambda i: (1, i, 0)),
        ],
        out_specs=pl.BlockSpec((_R, D_OUT), lambda i: (i, 0)),
        out_shape=jax.ShapeDtypeStruct((N, D_OUT), jnp.float32),
    )(q, parts1, parts1, parts0, parts0)


def kernel(x, edge_index, W_self0, W_neigh0, b0, W_self1, W_neigh1, b1):
    src = edge_index[0]
    dst = edge_index[1]
    x_ext = jnp.concatenate(
        [x, jnp.ones((N, 16), jnp.float32)], axis=1)
    parts0 = _segsum144(x_ext, src, dst).reshape(NC, NP, D_IN + 16)
    hs = _pre(x, W_self0, b0.reshape(1, -1))
    p, q = _dense0(hs, parts0, W_neigh0, W_neigh1, W_self1,
                   b1.reshape(1, -1))
    parts1 = _segsum128(p, src, dst).reshape(NC, NP, D_OUT)
    out = _dense1(q, parts1, parts0)
    return out
